# Initial kernel scaffold; baseline (speedup 1.0000x reference)
import jax
import jax.numpy as jnp
from jax.experimental import pallas as pl


def kernel(x1_x, x1_edge_index, x1_edge_attr, x1_direction, x1_target1_index, x1_target2_index, x1_batch, x1_direction2, x2_x, x2_edge_index, x2_batch, emb1, emb2, mlp1_W1, mlp1_b1, mlp1_W2, mlp1_b2, bn1_g, bn1_b, Wq1, Wk1, Wv1, mlp2_W1, mlp2_b1, mlp2_W2, mlp2_b2, bn2_g, bn2_b, Wq2, Wk2, Wv2, fc11_W, fc11_b, fc12_W, fc12_b):
    raise NotImplementedError("write your pallas kernel here")


# speedup vs baseline: 2.2896x; 2.2896x over previous
import functools

import jax
import jax.numpy as jnp
import numpy as np
from jax import lax
from jax.experimental import pallas as pl
from jax.experimental.pallas import tpu as pltpu
from jax.experimental.pallas import tpu_sc as plsc

NN = 50000
EE = 800000
BB = 128
DIM = 64
HEAD = 8
HD = DIM // HEAD
NLAYER = 4

_HI = lax.Precision.HIGHEST
_LO = lax.Precision.DEFAULT

NP = 50176
RB = 3136
NBLK = NP // RB
_BN_C = np.float32(np.sqrt(np.float32(1.0 + 1e-5)))
SCALE = 1.0 / np.sqrt(HD)


def _expander():
    r = lax.broadcasted_iota(jnp.int32, (HEAD, DIM), 0)
    c = lax.broadcasted_iota(jnp.int32, (HEAD, DIM), 1)
    return (c // HD == r).astype(jnp.float32)


def _reducer():
    r = lax.broadcasted_iota(jnp.int32, (DIM, HEAD), 0)
    c = lax.broadcasted_iota(jnp.int32, (DIM, HEAD), 1)
    return (r // HD == c).astype(jnp.float32)


def _emb_body(x_ref, emb_ref, out_ref):
    ids = lax.broadcasted_iota(jnp.int32, (RB, 128), 1).astype(jnp.float32)
    oh = (x_ref[...] == ids).astype(jnp.float32)
    out_ref[...] = jnp.dot(oh, emb_ref[...],
                           preferred_element_type=jnp.float32, precision=_HI)


def _emb_lookup(x_col, emb_pad):
    return pl.pallas_call(
        _emb_body,
        grid=(NBLK,),
        in_specs=[
            pl.BlockSpec((RB, 1), lambda i: (i, 0)),
            pl.BlockSpec((128, DIM), lambda i: (0, 0)),
        ],
        out_specs=pl.BlockSpec((RB, DIM), lambda i: (i, 0)),
        out_shape=jax.ShapeDtypeStruct((NP, DIM), jnp.float32),
    )(x_col, emb_pad)


def _mlp_body(prev_ref, z_ref, w1_ref, b1_ref, w2_ref, b2_ref, g_ref, bb_ref,
              out_ref):
    z = z_ref[...]
    h = jnp.maximum(
        jnp.dot(z, w1_ref[...], preferred_element_type=jnp.float32,
                precision=_LO)
        + b1_ref[...], 0.0)
    h = jnp.dot(h, w2_ref[...], preferred_element_type=jnp.float32,
                precision=_LO) + b2_ref[...]
    h = g_ref[...] * h / _BN_C + bb_ref[...]
    out_ref[...] = prev_ref[...] + h


def _mlp_layer(prev, z, w1, b1, w2, b2, g, bb):
    wspec = pl.BlockSpec((DIM, DIM), lambda i: (0, 0))
    vspec = pl.BlockSpec((1, DIM), lambda i: (0, 0))
    nspec = pl.BlockSpec((RB, DIM), lambda i: (i, 0))
    return pl.pallas_call(
        _mlp_body,
        grid=(NBLK,),
        in_specs=[nspec, nspec, wspec, vspec, wspec, vspec, vspec, vspec],
        out_specs=nspec,
        out_shape=jax.ShapeDtypeStruct((NP, DIM), jnp.float32),
    )(prev, z, w1, b1.reshape(1, DIM), w2, b2.reshape(1, DIM),
      g.reshape(1, DIM), bb.reshape(1, DIM))


def _passA_body(o1_ref, o2_ref, dirn_ref, t1_ref, t2_ref,
                wq1_ref, wk1_ref, wv1_ref, wq2_ref, wk2_ref, wv2_ref,
                q1_ref, v1_ref, q2_ref, v2_ref, ks_ref, acc):
    i = pl.program_id(0)
    o1 = jnp.concatenate([o1_ref[...], dirn_ref[...]], axis=1)
    o2 = jnp.concatenate([o2_ref[...], dirn_ref[...]], axis=1)

    q1_ref[...] = jnp.dot(o1, wq1_ref[...], preferred_element_type=jnp.float32,
                          precision=_LO)
    v1_ref[...] = jnp.dot(o1, wv1_ref[...], preferred_element_type=jnp.float32,
                          precision=_LO)
    q2_ref[...] = jnp.dot(o2, wq2_ref[...], preferred_element_type=jnp.float32,
                          precision=_LO)
    v2_ref[...] = jnp.dot(o2, wv2_ref[...], preferred_element_type=jnp.float32,
                          precision=_LO)

    @pl.when(i == 0)
    def _():
        acc[...] = jnp.zeros_like(acc)

    rows = (i * RB + lax.broadcasted_iota(jnp.int32, (RB, BB), 0)
            ).astype(jnp.float32)
    oh1 = (rows == t1_ref[...]).astype(jnp.float32)
    oh2 = (rows == t2_ref[...]).astype(jnp.float32)
    dn = (((0,), (0,)), ((), ()))
    acc[0] += lax.dot_general(oh1, o1, dn, preferred_element_type=jnp.float32, precision=_HI)
    acc[1] += lax.dot_general(oh2, o1, dn, preferred_element_type=jnp.float32, precision=_HI)
    acc[2] += lax.dot_general(oh1, o2, dn, preferred_element_type=jnp.float32, precision=_HI)
    acc[3] += lax.dot_general(oh2, o2, dn, preferred_element_type=jnp.float32, precision=_HI)

    ks_ref[0] = jnp.dot(acc[0], wk1_ref[...], preferred_element_type=jnp.float32,
                        precision=_LO)
    ks_ref[1] = jnp.dot(acc[1], wk1_ref[...], preferred_element_type=jnp.float32,
                        precision=_LO)
    ks_ref[2] = jnp.dot(acc[2], wk2_ref[...], preferred_element_type=jnp.float32,
                        precision=_LO)
    ks_ref[3] = jnp.dot(acc[3], wk2_ref[...], preferred_element_type=jnp.float32,
                        precision=_LO)


def _passA(out1, out2, dirn_pad, t1c, t2c, wq1, wk1, wv1, wq2, wk2, wv2):
    nspec = pl.BlockSpec((RB, DIM), lambda i: (i, 0))
    wspec = pl.BlockSpec((DIM + 1, DIM), lambda i: (0, 0))
    tspec = pl.BlockSpec((1, BB), lambda i: (0, 0))
    ospec = pl.BlockSpec((RB, DIM), lambda i: (i, 0))
    oshape = jax.ShapeDtypeStruct((NP, DIM), jnp.float32)
    return pl.pallas_call(
        _passA_body,
        grid=(NBLK,),
        in_specs=[nspec, nspec,
                  pl.BlockSpec((RB, 1), lambda i: (i, 0)), tspec, tspec,
                  wspec, wspec, wspec, wspec, wspec, wspec],
        out_specs=[ospec, ospec, ospec, ospec,
                   pl.BlockSpec((4, BB, DIM), lambda i: (0, 0, 0))],
        out_shape=[oshape, oshape, oshape, oshape,
                   jax.ShapeDtypeStruct((4, BB, DIM), jnp.float32)],
        scratch_shapes=[pltpu.VMEM((4, BB, DIM + 1), jnp.float32)],
    )(out1, out2, dirn_pad, t1c, t2c, wq1, wk1, wv1, wq2, wk2, wv2)


def _flash_body(q_ref, v_ref, k1_ref, k2_ref, g1_ref, g2_ref, sg_ref,
                s_out, a1_out, a2_out, m_s, s_s, a1_s, a2_s):
    i = pl.program_id(0)

    @pl.when(i == 0)
    def _():
        m_s[...] = jnp.full_like(m_s, -1e30)
        s_s[...] = jnp.zeros_like(s_s)
        a1_s[...] = jnp.zeros_like(a1_s)
        a2_s[...] = jnp.zeros_like(a2_s)

    ids = lax.broadcasted_iota(jnp.int32, (RB, BB), 1).astype(jnp.float32)
    ohg1 = (g1_ref[...] == ids).astype(jnp.float32)
    ohg2 = (g2_ref[...] == ids).astype(jnp.float32)
    ohsb = sg_ref[...] == ids
    ohs = ohsb.astype(jnp.float32)

    q = q_ref[...].astype(jnp.bfloat16).astype(jnp.float32)
    red = _reducer()
    kg1 = jnp.dot(ohg1, k1_ref[...], preferred_element_type=jnp.float32,
                  precision=_HI).astype(jnp.bfloat16).astype(jnp.float32)
    kg2 = jnp.dot(ohg2, k2_ref[...], preferred_element_type=jnp.float32,
                  precision=_HI).astype(jnp.bfloat16).astype(jnp.float32)
    l1 = jnp.dot(q * kg1, red, preferred_element_type=jnp.float32,
                 precision=_HI) * SCALE
    l2 = jnp.dot(q * kg2, red, preferred_element_type=jnp.float32,
                 precision=_HI) * SCALE
    L = jnp.concatenate([l1, l2], axis=1)

    mcols = []
    for c in range(2 * HEAD):
        tmp = jnp.where(ohsb, L[:, c:c + 1], -1e30)
        mcols.append(jnp.max(tmp, axis=0, keepdims=True))
    m_blk = jnp.concatenate(mcols, axis=0).T

    m_old = m_s[...]
    m_new = jnp.maximum(m_old, m_blk)
    alpha = jnp.exp(m_old - m_new)
    mg = jnp.dot(ohs, m_new, preferred_element_type=jnp.float32, precision=_HI)
    Ev = jnp.exp(L - mg)

    dn = (((0,), (0,)), ((), ()))
    exp8 = _expander()
    s_s[...] = s_s[...] * alpha + lax.dot_general(
        ohs, Ev, dn, preferred_element_type=jnp.float32, precision=_HI)
    vblk = v_ref[...]
    w1 = jnp.dot(Ev[:, :HEAD], exp8, preferred_element_type=jnp.float32, precision=_HI) * vblk
    w2 = jnp.dot(Ev[:, HEAD:], exp8, preferred_element_type=jnp.float32, precision=_HI) * vblk
    a1_s[...] = a1_s[...] * jnp.dot(alpha[:, :HEAD], exp8,
                                    preferred_element_type=jnp.float32, precision=_HI) \
        + lax.dot_general(ohs, w1, dn, preferred_element_type=jnp.float32, precision=_HI)
    a2_s[...] = a2_s[...] * jnp.dot(alpha[:, HEAD:], exp8,
                                    preferred_element_type=jnp.float32, precision=_HI) \
        + lax.dot_general(ohs, w2, dn, preferred_element_type=jnp.float32, precision=_HI)
    m_s[...] = m_new

    s_out[...] = s_s[...]
    a1_out[...] = a1_s[...]
    a2_out[...] = a2_s[...]


def _flash(qa, va, k1a, k2a, g1c, g2c, sgc):
    nspec = pl.BlockSpec((RB, DIM), lambda i: (i, 0))
    kspec = pl.BlockSpec((BB, DIM), lambda i: (0, 0))
    ispec = pl.BlockSpec((RB, 1), lambda i: (i, 0))
    return pl.pallas_call(
        _flash_body,
        grid=(NBLK,),
        in_specs=[nspec, nspec, kspec, kspec, ispec, ispec, ispec],
        out_specs=[pl.BlockSpec((BB, 2 * HEAD), lambda i: (0, 0)),
                   kspec, kspec],
        out_shape=[jax.ShapeDtypeStruct((BB, 2 * HEAD), jnp.float32),
                   jax.ShapeDtypeStruct((BB, DIM), jnp.float32),
                   jax.ShapeDtypeStruct((BB, DIM), jnp.float32)],
        scratch_shapes=[pltpu.VMEM((BB, 2 * HEAD), jnp.float32),
                        pltpu.VMEM((BB, 2 * HEAD), jnp.float32),
                        pltpu.VMEM((BB, DIM), jnp.float32),
                        pltpu.VMEM((BB, DIM), jnp.float32)],
    )(qa, va, k1a, k2a, g1c, g2c, sgc)


def _final_body(s1_ref, a11_ref, a21_ref, s2_ref, a12_ref, a22_ref, dir2_ref,
                w11_ref, b11_ref, w12_ref, b12_ref, out_ref):
    exp8 = _expander()

    def pooled(s_ref, a1_ref, a2_ref):
        s = s_ref[...]
        d1 = jnp.dot(s[:, :HEAD], exp8, preferred_element_type=jnp.float32, precision=_HI)
        d2 = jnp.dot(s[:, HEAD:], exp8, preferred_element_type=jnp.float32, precision=_HI)
        return (a1_ref[...] / jnp.maximum(d1, 1e-16)
                + a2_ref[...] / jnp.maximum(d2, 1e-16))

    h = pooled(s1_ref, a11_ref, a21_ref) + pooled(s2_ref, a12_ref, a22_ref)
    out = jnp.concatenate([h, dir2_ref[...]], axis=1)
    hid = jnp.dot(out, w11_ref[...], preferred_element_type=jnp.float32,
                  precision=_LO) + b11_ref[...]
    hid = 0.5 * hid * (1.0 + lax.erf(hid / np.sqrt(2.0)))
    o = jnp.dot(hid, w12_ref[...], preferred_element_type=jnp.float32,
                precision=_LO) + b12_ref[...]
    out_ref[...] = jax.nn.sigmoid(o)


def _final(s1, a11, a21, s2, a12, a22, dir2, w11, b11, w12, b12):
    return pl.pallas_call(
        _final_body,
        out_shape=jax.ShapeDtypeStruct((BB, 1), jnp.float32),
    )(s1, a11, a21, s2, a12, a22, dir2, w11, b11.reshape(1, DIM // 2),
      w12, b12.reshape(1, 1))


def _edge_z(out64, s, d, ea):
    msg = out64[s]
    if ea is not None:
        msg = jax.nn.relu(msg + ea)
    agg = jax.ops.segment_sum(msg, d, num_segments=NP)
    return out64 + agg


def _cat_w(w):
    return jnp.transpose(w, (1, 0, 2)).reshape(DIM + 1, DIM)


def kernel(x1_x, x1_edge_index, x1_edge_attr, x1_direction, x1_target1_index,
           x1_target2_index, x1_batch, x1_direction2, x2_x, x2_edge_index,
           x2_batch, emb1, emb2,
           mlp1_W1, mlp1_b1, mlp1_W2, mlp1_b2, bn1_g, bn1_b, Wq1, Wk1, Wv1,
           mlp2_W1, mlp2_b1, mlp2_W2, mlp2_b2, bn2_g, bn2_b, Wq2, Wk2, Wv2,
           fc11_W, fc11_b, fc12_W, fc12_b):
    pad_n = NP - NN

    def col_f32(a, pad_val):
        return jnp.pad(a.astype(jnp.float32), (0, pad_n),
                       constant_values=pad_val).reshape(NP, 1)

    x1c = col_f32(x1_x, 0)
    x2c = col_f32(x2_x, 0)
    b1c = col_f32(x1_batch, BB)
    b2c = col_f32(x2_batch, BB)
    t1c = x1_target1_index.astype(jnp.float32).reshape(1, BB)
    t2c = x1_target2_index.astype(jnp.float32).reshape(1, BB)
    emb1p = jnp.pad(emb1, ((0, 28), (0, 0)))
    emb2p = jnp.pad(emb2, ((0, 28), (0, 0)))
    dirn_pad = jnp.pad(x1_direction, ((0, pad_n), (0, 0)))

    s1 = x1_edge_index[0].astype(jnp.int32)
    d1 = x1_edge_index[1].astype(jnp.int32)
    s2 = x2_edge_index[0].astype(jnp.int32)
    d2 = x2_edge_index[1].astype(jnp.int32)

    out1 = _emb_lookup(x1c, emb1p)
    out2 = _emb_lookup(x2c, emb2p)

    for i in range(NLAYER):
        z = _edge_z(out1, s1, d1, x1_edge_attr)
        out1 = _mlp_layer(out1, z, mlp1_W1[i], mlp1_b1[i], mlp1_W2[i],
                          mlp1_b2[i], bn1_g[i], bn1_b[i])
    for i in range(NLAYER):
        z = _edge_z(out2, s2, d2, None)
        out2 = _mlp_layer(out2, z, mlp2_W1[i], mlp2_b1[i], mlp2_W2[i],
                          mlp2_b2[i], bn2_g[i], bn2_b[i])

    qa1, va1, qa2, va2, ks = _passA(
        out1, out2, dirn_pad, t1c, t2c,
        _cat_w(Wq1), _cat_w(Wk1), _cat_w(Wv1),
        _cat_w(Wq2), _cat_w(Wk2), _cat_w(Wv2))

    sA, a1A, a2A = _flash(qa1, va1, ks[0], ks[1], b1c, b1c, b1c)
    sB, a1B, a2B = _flash(qa2, va2, ks[2], ks[3], b1c, b2c, b2c)

    return _final(sA, a1A, a2A, sB, a1B, a2B, x1_direction2,
                  fc11_W, fc11_b, fc12_W, fc12_b)


# speedup vs baseline: 3.5159x; 1.5355x over previous
import functools

import jax
import jax.numpy as jnp
import numpy as np
from jax import lax
from jax.experimental import pallas as pl
from jax.experimental.pallas import tpu as pltpu
from jax.experimental.pallas import tpu_sc as plsc

NN = 50000
EE = 800000
BB = 128
DIM = 64
HEAD = 8
HD = DIM // HEAD
NLAYER = 4

_HI = lax.Precision.HIGHEST
_LO = lax.Precision.DEFAULT

NP = 50176
RB = 3136
NBLK = NP // RB
_BN_C = np.float32(np.sqrt(np.float32(1.0 + 1e-5)))
SCALE = 1.0 / np.sqrt(HD)


def _expander():
    r = lax.broadcasted_iota(jnp.int32, (HEAD, DIM), 0)
    c = lax.broadcasted_iota(jnp.int32, (HEAD, DIM), 1)
    return (c // HD == r).astype(jnp.float32)


def _reducer():
    r = lax.broadcasted_iota(jnp.int32, (DIM, HEAD), 0)
    c = lax.broadcasted_iota(jnp.int32, (DIM, HEAD), 1)
    return (r // HD == c).astype(jnp.float32)


def _emb_body(x_ref, emb_ref, out_ref):
    ids = lax.broadcasted_iota(jnp.int32, (RB, 128), 1).astype(jnp.float32)
    oh = (x_ref[...] == ids).astype(jnp.float32)
    out_ref[...] = jnp.dot(oh, emb_ref[...],
                           preferred_element_type=jnp.float32, precision=_HI)


def _emb_lookup(x_col, emb_pad):
    return pl.pallas_call(
        _emb_body,
        grid=(NBLK,),
        in_specs=[
            pl.BlockSpec((RB, 1), lambda i: (i, 0)),
            pl.BlockSpec((128, DIM), lambda i: (0, 0)),
        ],
        out_specs=pl.BlockSpec((RB, DIM), lambda i: (i, 0)),
        out_shape=jax.ShapeDtypeStruct((NP, DIM), jnp.float32),
    )(x_col, emb_pad)


def _mlp_body(prev_ref, z_ref, w1_ref, b1_ref, w2_ref, b2_ref, g_ref, bb_ref,
              out_ref):
    z = z_ref[...]
    h = jnp.maximum(
        jnp.dot(z, w1_ref[...], preferred_element_type=jnp.float32,
                precision=_LO)
        + b1_ref[...], 0.0)
    h = jnp.dot(h, w2_ref[...], preferred_element_type=jnp.float32,
                precision=_LO) + b2_ref[...]
    h = g_ref[...] * h / _BN_C + bb_ref[...]
    out_ref[...] = prev_ref[...] + h


def _mlp_layer(prev, z, w1, b1, w2, b2, g, bb):
    wspec = pl.BlockSpec((DIM, DIM), lambda i: (0, 0))
    vspec = pl.BlockSpec((1, DIM), lambda i: (0, 0))
    nspec = pl.BlockSpec((RB, DIM), lambda i: (i, 0))
    return pl.pallas_call(
        _mlp_body,
        grid=(NBLK,),
        in_specs=[nspec, nspec, wspec, vspec, wspec, vspec, vspec, vspec],
        out_specs=nspec,
        out_shape=jax.ShapeDtypeStruct((NP, DIM), jnp.float32),
    )(prev, z, w1, b1.reshape(1, DIM), w2, b2.reshape(1, DIM),
      g.reshape(1, DIM), bb.reshape(1, DIM))


def _passA_body(o1_ref, o2_ref, dirn_ref, t1_ref, t2_ref,
                wq1_ref, wk1_ref, wv1_ref, wq2_ref, wk2_ref, wv2_ref,
                q1_ref, v1_ref, q2_ref, v2_ref, ks_ref, acc):
    i = pl.program_id(0)
    o1 = jnp.concatenate([o1_ref[...], dirn_ref[...]], axis=1)
    o2 = jnp.concatenate([o2_ref[...], dirn_ref[...]], axis=1)

    q1_ref[...] = jnp.dot(o1, wq1_ref[...], preferred_element_type=jnp.float32,
                          precision=_LO)
    v1_ref[...] = jnp.dot(o1, wv1_ref[...], preferred_element_type=jnp.float32,
                          precision=_LO)
    q2_ref[...] = jnp.dot(o2, wq2_ref[...], preferred_element_type=jnp.float32,
                          precision=_LO)
    v2_ref[...] = jnp.dot(o2, wv2_ref[...], preferred_element_type=jnp.float32,
                          precision=_LO)

    @pl.when(i == 0)
    def _():
        acc[...] = jnp.zeros_like(acc)

    rows = (i * RB + lax.broadcasted_iota(jnp.int32, (RB, BB), 0)
            ).astype(jnp.float32)
    oh1 = (rows == t1_ref[...]).astype(jnp.float32)
    oh2 = (rows == t2_ref[...]).astype(jnp.float32)
    dn = (((0,), (0,)), ((), ()))
    acc[0] += lax.dot_general(oh1, o1, dn, preferred_element_type=jnp.float32, precision=_HI)
    acc[1] += lax.dot_general(oh2, o1, dn, preferred_element_type=jnp.float32, precision=_HI)
    acc[2] += lax.dot_general(oh1, o2, dn, preferred_element_type=jnp.float32, precision=_HI)
    acc[3] += lax.dot_general(oh2, o2, dn, preferred_element_type=jnp.float32, precision=_HI)

    ks_ref[0] = jnp.dot(acc[0], wk1_ref[...], preferred_element_type=jnp.float32,
                        precision=_LO)
    ks_ref[1] = jnp.dot(acc[1], wk1_ref[...], preferred_element_type=jnp.float32,
                        precision=_LO)
    ks_ref[2] = jnp.dot(acc[2], wk2_ref[...], preferred_element_type=jnp.float32,
                        precision=_LO)
    ks_ref[3] = jnp.dot(acc[3], wk2_ref[...], preferred_element_type=jnp.float32,
                        precision=_LO)


def _passA(out1, out2, dirn_pad, t1c, t2c, wq1, wk1, wv1, wq2, wk2, wv2):
    nspec = pl.BlockSpec((RB, DIM), lambda i: (i, 0))
    wspec = pl.BlockSpec((DIM + 1, DIM), lambda i: (0, 0))
    tspec = pl.BlockSpec((1, BB), lambda i: (0, 0))
    ospec = pl.BlockSpec((RB, DIM), lambda i: (i, 0))
    oshape = jax.ShapeDtypeStruct((NP, DIM), jnp.float32)
    return pl.pallas_call(
        _passA_body,
        grid=(NBLK,),
        in_specs=[nspec, nspec,
                  pl.BlockSpec((RB, 1), lambda i: (i, 0)), tspec, tspec,
                  wspec, wspec, wspec, wspec, wspec, wspec],
        out_specs=[ospec, ospec, ospec, ospec,
                   pl.BlockSpec((4, BB, DIM), lambda i: (0, 0, 0))],
        out_shape=[oshape, oshape, oshape, oshape,
                   jax.ShapeDtypeStruct((4, BB, DIM), jnp.float32)],
        scratch_shapes=[pltpu.VMEM((4, BB, DIM + 1), jnp.float32)],
    )(out1, out2, dirn_pad, t1c, t2c, wq1, wk1, wv1, wq2, wk2, wv2)


def _flash_body(q_ref, v_ref, k1_ref, k2_ref, g1_ref, g2_ref, sg_ref,
                s_out, a1_out, a2_out, m_s, s_s, a1_s, a2_s):
    i = pl.program_id(0)

    @pl.when(i == 0)
    def _():
        m_s[...] = jnp.full_like(m_s, -1e30)
        s_s[...] = jnp.zeros_like(s_s)
        a1_s[...] = jnp.zeros_like(a1_s)
        a2_s[...] = jnp.zeros_like(a2_s)

    ids = lax.broadcasted_iota(jnp.int32, (RB, BB), 1).astype(jnp.float32)
    ohg1 = (g1_ref[...] == ids).astype(jnp.float32)
    ohg2 = (g2_ref[...] == ids).astype(jnp.float32)
    ohsb = sg_ref[...] == ids
    ohs = ohsb.astype(jnp.float32)

    q = q_ref[...].astype(jnp.bfloat16).astype(jnp.float32)
    red = _reducer()
    kg1 = jnp.dot(ohg1, k1_ref[...], preferred_element_type=jnp.float32,
                  precision=_HI).astype(jnp.bfloat16).astype(jnp.float32)
    kg2 = jnp.dot(ohg2, k2_ref[...], preferred_element_type=jnp.float32,
                  precision=_HI).astype(jnp.bfloat16).astype(jnp.float32)
    l1 = jnp.dot(q * kg1, red, preferred_element_type=jnp.float32,
                 precision=_HI) * SCALE
    l2 = jnp.dot(q * kg2, red, preferred_element_type=jnp.float32,
                 precision=_HI) * SCALE
    L = jnp.concatenate([l1, l2], axis=1)

    mcols = []
    for c in range(2 * HEAD):
        tmp = jnp.where(ohsb, L[:, c:c + 1], -1e30)
        mcols.append(jnp.max(tmp, axis=0, keepdims=True))
    m_blk = jnp.concatenate(mcols, axis=0).T

    m_old = m_s[...]
    m_new = jnp.maximum(m_old, m_blk)
    alpha = jnp.exp(m_old - m_new)
    mg = jnp.dot(ohs, m_new, preferred_element_type=jnp.float32, precision=_HI)
    Ev = jnp.exp(L - mg)

    dn = (((0,), (0,)), ((), ()))
    exp8 = _expander()
    s_s[...] = s_s[...] * alpha + lax.dot_general(
        ohs, Ev, dn, preferred_element_type=jnp.float32, precision=_HI)
    vblk = v_ref[...]
    w1 = jnp.dot(Ev[:, :HEAD], exp8, preferred_element_type=jnp.float32, precision=_HI) * vblk
    w2 = jnp.dot(Ev[:, HEAD:], exp8, preferred_element_type=jnp.float32, precision=_HI) * vblk
    a1_s[...] = a1_s[...] * jnp.dot(alpha[:, :HEAD], exp8,
                                    preferred_element_type=jnp.float32, precision=_HI) \
        + lax.dot_general(ohs, w1, dn, preferred_element_type=jnp.float32, precision=_HI)
    a2_s[...] = a2_s[...] * jnp.dot(alpha[:, HEAD:], exp8,
                                    preferred_element_type=jnp.float32, precision=_HI) \
        + lax.dot_general(ohs, w2, dn, preferred_element_type=jnp.float32, precision=_HI)
    m_s[...] = m_new

    s_out[...] = s_s[...]
    a1_out[...] = a1_s[...]
    a2_out[...] = a2_s[...]


def _flash(qa, va, k1a, k2a, g1c, g2c, sgc):
    nspec = pl.BlockSpec((RB, DIM), lambda i: (i, 0))
    kspec = pl.BlockSpec((BB, DIM), lambda i: (0, 0))
    ispec = pl.BlockSpec((RB, 1), lambda i: (i, 0))
    return pl.pallas_call(
        _flash_body,
        grid=(NBLK,),
        in_specs=[nspec, nspec, kspec, kspec, ispec, ispec, ispec],
        out_specs=[pl.BlockSpec((BB, 2 * HEAD), lambda i: (0, 0)),
                   kspec, kspec],
        out_shape=[jax.ShapeDtypeStruct((BB, 2 * HEAD), jnp.float32),
                   jax.ShapeDtypeStruct((BB, DIM), jnp.float32),
                   jax.ShapeDtypeStruct((BB, DIM), jnp.float32)],
        scratch_shapes=[pltpu.VMEM((BB, 2 * HEAD), jnp.float32),
                        pltpu.VMEM((BB, 2 * HEAD), jnp.float32),
                        pltpu.VMEM((BB, DIM), jnp.float32),
                        pltpu.VMEM((BB, DIM), jnp.float32)],
    )(qa, va, k1a, k2a, g1c, g2c, sgc)


def _final_body(s1_ref, a11_ref, a21_ref, s2_ref, a12_ref, a22_ref, dir2_ref,
                w11_ref, b11_ref, w12_ref, b12_ref, out_ref):
    exp8 = _expander()

    def pooled(s_ref, a1_ref, a2_ref):
        s = s_ref[...]
        d1 = jnp.dot(s[:, :HEAD], exp8, preferred_element_type=jnp.float32, precision=_HI)
        d2 = jnp.dot(s[:, HEAD:], exp8, preferred_element_type=jnp.float32, precision=_HI)
        return (a1_ref[...] / jnp.maximum(d1, 1e-16)
                + a2_ref[...] / jnp.maximum(d2, 1e-16))

    h = pooled(s1_ref, a11_ref, a21_ref) + pooled(s2_ref, a12_ref, a22_ref)
    out = jnp.concatenate([h, dir2_ref[...]], axis=1)
    hid = jnp.dot(out, w11_ref[...], preferred_element_type=jnp.float32,
                  precision=_LO) + b11_ref[...]
    hid = 0.5 * hid * (1.0 + lax.erf(hid / np.sqrt(2.0)))
    o = jnp.dot(hid, w12_ref[...], preferred_element_type=jnp.float32,
                precision=_LO) + b12_ref[...]
    out_ref[...] = jax.nn.sigmoid(o)


def _final(s1, a11, a21, s2, a12, a22, dir2, w11, b11, w12, b12):
    return pl.pallas_call(
        _final_body,
        out_shape=jax.ShapeDtypeStruct((BB, 1), jnp.float32),
    )(s1, a11, a21, s2, a12, a22, dir2, w11, b11.reshape(1, DIM // 2),
      w12, b12.reshape(1, 1))


ECH = EE // 128
CPS = (ECH + 15) // 16
STRIPE = NP // 16

_SC_MESH = plsc.VectorSubcoreMesh(core_axis_name="c", subcore_axis_name="s",
                                  num_cores=2, num_subcores=16)


def _sc_body_noea(out_hbm, s_hbm, d_hbm, z_hbm, sidx, didx, gbuf, sem, accum):
    c = lax.axis_index("c")
    sid = lax.axis_index("s")
    pltpu.sync_copy(out_hbm.at[c, pl.ds(sid * STRIPE, STRIPE)],
                    accum.at[pl.ds(sid * STRIPE, STRIPE)])
    plsc.subcore_barrier()

    def chunk(t, carry):
        ch = sid * CPS + t

        @pl.when(ch < ECH)
        def _():
            pltpu.sync_copy(s_hbm.at[ch], sidx)
            pltpu.sync_copy(d_hbm.at[ch], didx)
            pltpu.async_copy(out_hbm.at[c].at[sidx], gbuf, sem).wait()
            pltpu.sync_copy(gbuf, accum.at[didx], add=True)

        return carry

    lax.fori_loop(0, CPS, chunk, 0)
    plsc.subcore_barrier()
    pltpu.sync_copy(accum.at[pl.ds(sid * STRIPE, STRIPE)],
                    z_hbm.at[c, pl.ds(sid * STRIPE, STRIPE)])


def _sc_body_ea(out_hbm, s_hbm, d_hbm, ea_hbm, z_hbm, sidx, didx, gbuf, eabuf,
                sem, accum):
    c = lax.axis_index("c")
    sid = lax.axis_index("s")
    pltpu.sync_copy(out_hbm.at[c, pl.ds(sid * STRIPE, STRIPE)],
                    accum.at[pl.ds(sid * STRIPE, STRIPE)])
    plsc.subcore_barrier()

    def chunk(t, carry):
        ch = sid * CPS + t

        @pl.when(ch < ECH)
        def _():
            pltpu.sync_copy(s_hbm.at[ch], sidx)
            pltpu.sync_copy(d_hbm.at[ch], didx)
            pltpu.async_copy(out_hbm.at[c].at[sidx], gbuf, sem).wait()
            pltpu.sync_copy(
                ea_hbm.at[pl.ds(ch * 128, 128), pl.ds(c * 32, 32)], eabuf)

            def vrow(k, cc):
                r = k // 2
                co = (k % 2) * 16
                v = gbuf[r, pl.ds(co, 16)] + eabuf[r, pl.ds(co, 16)]
                gbuf[r, pl.ds(co, 16)] = jnp.maximum(v, 0.0)
                return cc

            lax.fori_loop(0, 256, vrow, 0)
            pltpu.sync_copy(gbuf, accum.at[didx], add=True)

        return carry

    lax.fori_loop(0, CPS, chunk, 0)
    plsc.subcore_barrier()
    pltpu.sync_copy(accum.at[pl.ds(sid * STRIPE, STRIPE)],
                    z_hbm.at[c, pl.ds(sid * STRIPE, STRIPE)])


def _sc_edge_call(out_cat, s2d, d2d, ea):
    base_scr = [pltpu.VMEM((128,), jnp.int32),
                pltpu.VMEM((128,), jnp.int32),
                pltpu.VMEM((128, 32), jnp.float32)]
    tail_scr = [pltpu.SemaphoreType.DMA,
                pltpu.VMEM_SHARED((NP, 32), jnp.float32)]
    cparams = pltpu.CompilerParams(use_tc_tiling_on_sc=False)
    if ea is None:
        fn = pl.kernel(
            _sc_body_noea,
            out_type=jax.ShapeDtypeStruct((2, NP, 32), jnp.float32),
            mesh=_SC_MESH,
            compiler_params=cparams,
            scratch_types=base_scr + tail_scr)
        return fn(out_cat, s2d, d2d)
    fn = pl.kernel(
        _sc_body_ea,
        out_type=jax.ShapeDtypeStruct((2, NP, 32), jnp.float32),
        mesh=_SC_MESH,
        compiler_params=cparams,
        scratch_types=base_scr + [pltpu.VMEM((128, 32), jnp.float32)]
        + tail_scr)
    return fn(out_cat, s2d, d2d, ea)


def _edge_z(out64, s2d, d2d, ea):
    out_cat = jnp.stack([out64[:, :32], out64[:, 32:]])
    z_cat = _sc_edge_call(out_cat, s2d, d2d, ea)
    return jnp.concatenate([z_cat[0], z_cat[1]], axis=1)


def _cat_w(w):
    return jnp.transpose(w, (1, 0, 2)).reshape(DIM + 1, DIM)


def kernel(x1_x, x1_edge_index, x1_edge_attr, x1_direction, x1_target1_index,
           x1_target2_index, x1_batch, x1_direction2, x2_x, x2_edge_index,
           x2_batch, emb1, emb2,
           mlp1_W1, mlp1_b1, mlp1_W2, mlp1_b2, bn1_g, bn1_b, Wq1, Wk1, Wv1,
           mlp2_W1, mlp2_b1, mlp2_W2, mlp2_b2, bn2_g, bn2_b, Wq2, Wk2, Wv2,
           fc11_W, fc11_b, fc12_W, fc12_b):
    pad_n = NP - NN

    def col_f32(a, pad_val):
        return jnp.pad(a.astype(jnp.float32), (0, pad_n),
                       constant_values=pad_val).reshape(NP, 1)

    x1c = col_f32(x1_x, 0)
    x2c = col_f32(x2_x, 0)
    b1c = col_f32(x1_batch, BB)
    b2c = col_f32(x2_batch, BB)
    t1c = x1_target1_index.astype(jnp.float32).reshape(1, BB)
    t2c = x1_target2_index.astype(jnp.float32).reshape(1, BB)
    emb1p = jnp.pad(emb1, ((0, 28), (0, 0)))
    emb2p = jnp.pad(emb2, ((0, 28), (0, 0)))
    dirn_pad = jnp.pad(x1_direction, ((0, pad_n), (0, 0)))

    s1 = x1_edge_index[0].astype(jnp.int32).reshape(ECH, 128)
    d1 = x1_edge_index[1].astype(jnp.int32).reshape(ECH, 128)
    s2 = x2_edge_index[0].astype(jnp.int32).reshape(ECH, 128)
    d2 = x2_edge_index[1].astype(jnp.int32).reshape(ECH, 128)

    out1 = _emb_lookup(x1c, emb1p)
    out2 = _emb_lookup(x2c, emb2p)

    for i in range(NLAYER):
        z = _edge_z(out1, s1, d1, x1_edge_attr)
        out1 = _mlp_layer(out1, z, mlp1_W1[i], mlp1_b1[i], mlp1_W2[i],
                          mlp1_b2[i], bn1_g[i], bn1_b[i])
    for i in range(NLAYER):
        z = _edge_z(out2, s2, d2, None)
        out2 = _mlp_layer(out2, z, mlp2_W1[i], mlp2_b1[i], mlp2_W2[i],
                          mlp2_b2[i], bn2_g[i], bn2_b[i])

    qa1, va1, qa2, va2, ks = _passA(
        out1, out2, dirn_pad, t1c, t2c,
        _cat_w(Wq1), _cat_w(Wk1), _cat_w(Wv1),
        _cat_w(Wq2), _cat_w(Wk2), _cat_w(Wv2))

    sA, a1A, a2A = _flash(qa1, va1, ks[0], ks[1], b1c, b1c, b1c)
    sB, a1B, a2B = _flash(qa2, va2, ks[2], ks[3], b1c, b2c, b2c)

    return _final(sA, a1A, a2A, sB, a1B, a2B, x1_direction2,
                  fc11_W, fc11_b, fc12_W, fc12_b)
